# baseline (speedup 1.0000x reference)
import functools

import jax
import jax.numpy as jnp
from jax import lax
from jax.experimental import pallas as pl
from jax.experimental.pallas import tpu as pltpu
from jax.experimental.pallas import tpu_sc as plsc

_VOCAB = 1_000_000
_EMBED = 64
_SCALE = 8.0
_ROWS = 16384
_SEQ = 20
_NC, _NS = 2, 16
_NW = _NC * _NS
_RPW = _ROWS // _NW
_OC = 16
_NBUF = 4
_NCHUNK = _RPW // _OC


_BV = 4096
_NBLK = (_VOCAB + _BV - 1) // _BV


def _relayout_body(t_ref, o_ref):
    o_ref[...] = t_ref[...].T * _SCALE


_relayout = pl.pallas_call(
    _relayout_body,
    out_shape=jax.ShapeDtypeStruct((_VOCAB, _EMBED), jnp.float32),
    grid=(_NBLK,),
    in_specs=[pl.BlockSpec((_EMBED, _BV), lambda i: (0, i))],
    out_specs=pl.BlockSpec((_BV, _EMBED), lambda i: (i, 0)),
)


_mesh = plsc.VectorSubcoreMesh(core_axis_name="c", subcore_axis_name="s")


@functools.partial(
    pl.kernel,
    out_type=jax.ShapeDtypeStruct((_ROWS, _SEQ, _EMBED), jnp.float32),
    mesh=_mesh,
    scratch_types=[
        pltpu.VMEM((_RPW, _SEQ), jnp.int32),
        pltpu.VMEM((_NBUF, _OC, _SEQ, _EMBED), jnp.float32),
    ] + [pltpu.SemaphoreType.DMA] * (2 * _NBUF),
    compiler_params=pltpu.CompilerParams(
        use_tc_tiling_on_sc=False, disable_bounds_checks=True),
)
def _embed_lookup(idx_hbm, table_hbm, out_hbm, idx_all, rows, *sems):
    gsems, wsems = sems[:_NBUF], sems[_NBUF:]
    wid = lax.axis_index("s") * _NC + lax.axis_index("c")
    base = wid * _RPW

    pltpu.sync_copy(idx_hbm.at[pl.ds(base, _RPW)], idx_all)

    def start_gathers(c, b):
        r0 = c * _OC
        for j in range(_OC):
            pltpu.async_copy(
                table_hbm.at[idx_all.at[r0 + j]], rows.at[b, j], gsems[b])

    def wait_gathers(b):
        pltpu.make_async_copy(
            out_hbm.at[pl.ds(0, _OC)], rows.at[b], gsems[b]).wait()

    def out_sl(c):
        return out_hbm.at[pl.ds(base + c * _OC, _OC)]

    def start_write(c, b):
        pltpu.async_copy(rows.at[b], out_sl(c), wsems[b])

    def wait_write(b):
        pltpu.make_async_copy(rows.at[b], out_sl(0), wsems[b]).wait()

    start_gathers(0, 0)
    start_gathers(1, 1)

    @pl.loop(0, _NCHUNK, step=_NBUF)
    def _round(base_c):
        for b in range(_NBUF):
            c = base_c + b
            wait_gathers(b)
            start_write(c, b)
            bp = (b + 2) % _NBUF

            @pl.when(c + 2 < _NCHUNK)
            def _prefetch():
                @pl.when(c >= 2)
                def _drain():
                    wait_write(bp)
                start_gathers(c + 2, bp)

    for b in range(_NBUF):
        wait_write(b)


def kernel(x, table):
    table_rm = _relayout(table.T)
    return _embed_lookup(x, table_rm)


# speedup vs baseline: 1.1485x; 1.1485x over previous
import functools

import jax
import jax.numpy as jnp
from jax import lax
from jax.experimental import pallas as pl
from jax.experimental.pallas import tpu as pltpu
from jax.experimental.pallas import tpu_sc as plsc

_VOCAB = 1_000_000
_EMBED = 64
_SCALE = 8.0
_ROWS = 16384
_SEQ = 20
_NC, _NS = 2, 16
_NW = _NC * _NS
_RPW = _ROWS // _NW
_OC = 16
_NBUF = 4
_NCHUNK = _RPW // _OC


_BV = 4096
_NBLK = (_VOCAB + _BV - 1) // _BV


def _relayout_body(t_ref, o_ref):
    o_ref[...] = t_ref[...].T * _SCALE


_relayout = pl.pallas_call(
    _relayout_body,
    out_shape=jax.ShapeDtypeStruct((_VOCAB, _EMBED), jnp.float32),
    grid=(_NBLK,),
    in_specs=[pl.BlockSpec((_EMBED, _BV), lambda i: (0, i))],
    out_specs=pl.BlockSpec((_BV, _EMBED), lambda i: (i, 0)),
)


_mesh = plsc.VectorSubcoreMesh(core_axis_name="c", subcore_axis_name="s")


@functools.partial(
    pl.kernel,
    out_type=jax.ShapeDtypeStruct((_ROWS, _SEQ, _EMBED), jnp.float32),
    mesh=_mesh,
    scratch_types=[
        pltpu.VMEM((_RPW, _SEQ), jnp.int32),
        pltpu.VMEM((_NBUF, _OC, _SEQ, _EMBED), jnp.float32),
    ] + [pltpu.SemaphoreType.DMA] * (2 * _NBUF),
    compiler_params=pltpu.CompilerParams(
        use_tc_tiling_on_sc=False, disable_bounds_checks=True,
        skip_device_barrier=True),
)
def _embed_lookup(idx_hbm, table_hbm, out_hbm, idx_all, rows, *sems):
    gsems, wsems = sems[:_NBUF], sems[_NBUF:]
    wid = lax.axis_index("s") * _NC + lax.axis_index("c")
    base = wid * _RPW

    pltpu.sync_copy(idx_hbm.at[pl.ds(base, _RPW)], idx_all)

    def start_gathers(c, b):
        r0 = c * _OC
        for j in range(_OC):
            pltpu.async_copy(
                table_hbm.at[idx_all.at[r0 + j]], rows.at[b, j], gsems[b])

    def wait_gathers(b):
        pltpu.make_async_copy(
            out_hbm.at[pl.ds(0, _OC)], rows.at[b], gsems[b]).wait()

    def out_sl(c):
        return out_hbm.at[pl.ds(base + c * _OC, _OC)]

    def start_write(c, b):
        pltpu.async_copy(rows.at[b], out_sl(c), wsems[b])

    def wait_write(b):
        pltpu.make_async_copy(rows.at[b], out_sl(0), wsems[b]).wait()

    start_gathers(0, 0)
    start_gathers(1, 1)

    @pl.loop(0, _NCHUNK, step=_NBUF)
    def _round(base_c):
        for b in range(_NBUF):
            c = base_c + b
            wait_gathers(b)

            @plsc.parallel_loop(0, _OC, unroll=2)
            def _scale(i):
                for j in range(_SEQ):
                    for k in range(_EMBED // 16):
                        sl = pl.ds(k * 16, 16)
                        rows[b, i, j, sl] = rows[b, i, j, sl] * _SCALE

            start_write(c, b)
            bp = (b + 2) % _NBUF

            @pl.when(c + 2 < _NCHUNK)
            def _prefetch():
                @pl.when(c >= 2)
                def _drain():
                    wait_write(bp)
                start_gathers(c + 2, bp)

    for b in range(_NBUF):
        wait_write(b)


def kernel(x, table):
    return _embed_lookup(x, table)
